# baseline (speedup 1.0000x reference)
import jax
import jax.numpy as jnp
from jax.experimental import pallas as pl
from jax.experimental.pallas import tpu as pltpu

_B, _N, _D, _O = 32, 512, 128, 128
_S = 8
_RB = 512
_LOG2E = 1.4426950408889634


def _fused_attention_kernel(x_ref, W_ref, asrc_ref, adst_ref, bias_ref, o_ref):
    f32 = jnp.float32
    W = W_ref[...]
    a_src = asrc_ref[...]
    a_dst = adst_ref[...]

    ones_col = jnp.ones((_N, 1), dtype=f32)
    bias_v = bias_ref[...]
    for i in range(_S):
        H = jnp.dot(x_ref[i], W, preferred_element_type=f32)
        H_bf = H.astype(jnp.bfloat16)
        d_col = jax.lax.dot_general(H, a_dst, (((1,), (1,)), ((), ())),
                                    preferred_element_type=f32)
        s_row = jax.lax.dot_general(a_src, H, (((1,), (1,)), ((), ())),
                                    preferred_element_type=f32)

        s_max = jnp.max(s_row, axis=1, keepdims=True)
        dm = d_col + s_max
        m_col = jnp.maximum(dm, 0.2 * dm)

        u_col = jnp.exp2(_LOG2E * (d_col - m_col)).astype(jnp.bfloat16)
        w_col = jnp.exp2(_LOG2E * (0.2 * d_col - m_col)).astype(jnp.bfloat16)
        v_row = jnp.exp2(_LOG2E * s_row).astype(jnp.bfloat16)
        z_row = jnp.exp2(0.2 * _LOG2E * s_row).astype(jnp.bfloat16)

        for r in range(_N // _RB):
            rs = slice(r * _RB, (r + 1) * _RB)
            p_bf = jnp.maximum(u_col[rs] * v_row, w_col[rs] * z_row)
            denom = jnp.dot(p_bf, ones_col, preferred_element_type=f32)
            num = jnp.dot(p_bf, H_bf, preferred_element_type=f32)
            out = num / denom + bias_v
            o_ref[i, rs] = jnp.where(out > 0, out, jnp.exp(out) - 1.0)


def kernel(x, W, a_src, a_dst, bias):
    grid = (_B // _S,)
    out = pl.pallas_call(
        _fused_attention_kernel,
        grid=grid,
        in_specs=[
            pl.BlockSpec((_S, _N, _D), lambda b: (b, 0, 0)),
            pl.BlockSpec((_D, _O), lambda b: (0, 0)),
            pl.BlockSpec((1, _O), lambda b: (0, 0)),
            pl.BlockSpec((1, _O), lambda b: (0, 0)),
            pl.BlockSpec((1, _O), lambda b: (0, 0)),
        ],
        out_specs=pl.BlockSpec((_S, _N, _O), lambda b: (b, 0, 0)),
        out_shape=jax.ShapeDtypeStruct((_B, _N, _O), jnp.float32),
        compiler_params=pltpu.CompilerParams(
            dimension_semantics=("parallel",)),
    )(x, W, a_src.reshape(1, _O), a_dst.reshape(1, _O), bias.reshape(1, _O))
    return out


# speedup vs baseline: 1.6485x; 1.6485x over previous
import jax
import jax.numpy as jnp
from jax.experimental import pallas as pl
from jax.experimental.pallas import tpu as pltpu

_B, _N, _D, _O = 32, 512, 128, 128
_S = 8
_RB = 512
_LOG2E = 1.4426950408889634


def _fused_attention_kernel(x_ref, W_ref, asrc_ref, adst_ref, bias_ref, o_ref):
    f32 = jnp.float32
    x = x_ref[...].reshape(_S * _N, _D)
    W = W_ref[...]
    H_all = jnp.dot(x, W, preferred_element_type=f32)

    a_src = asrc_ref[...]
    a_dst = adst_ref[...]
    d_all = jax.lax.dot_general(H_all, a_dst, (((1,), (1,)), ((), ())),
                                preferred_element_type=f32)
    s_all = jax.lax.dot_general(a_src, H_all, (((1,), (1,)), ((), ())),
                                preferred_element_type=f32)

    ones_col = jnp.ones((_N, 1), dtype=f32)
    bias_v = bias_ref[...]
    for i in range(_S):
        ss = slice(i * _N, (i + 1) * _N)
        H = H_all[ss]
        H_bf = H.astype(jnp.bfloat16)
        d_col = d_all[ss]
        s_row = s_all[:, ss]

        s_max = jnp.max(s_row, axis=1, keepdims=True)
        dm = d_col + s_max
        m_col = jnp.maximum(dm, 0.2 * dm)

        u_col = jnp.exp2(_LOG2E * (d_col - m_col)).astype(jnp.bfloat16)
        w_col = jnp.exp2(_LOG2E * (0.2 * d_col - m_col)).astype(jnp.bfloat16)
        v_row = jnp.exp2(_LOG2E * s_row).astype(jnp.bfloat16)
        z_row = jnp.exp2(0.2 * _LOG2E * s_row).astype(jnp.bfloat16)

        for r in range(_N // _RB):
            rs = slice(r * _RB, (r + 1) * _RB)
            p_bf = jnp.maximum(u_col[rs] * v_row, w_col[rs] * z_row)
            denom = jnp.dot(p_bf, ones_col, preferred_element_type=f32)
            num = jnp.dot(p_bf, H_bf, preferred_element_type=f32)
            out = num / denom + bias_v
            o_ref[i, rs] = jnp.where(out > 0, out, jnp.exp(out) - 1.0)


def kernel(x, W, a_src, a_dst, bias):
    grid = (_B // _S,)
    out = pl.pallas_call(
        _fused_attention_kernel,
        grid=grid,
        in_specs=[
            pl.BlockSpec((_S, _N, _D), lambda b: (b, 0, 0)),
            pl.BlockSpec((_D, _O), lambda b: (0, 0)),
            pl.BlockSpec((1, _O), lambda b: (0, 0)),
            pl.BlockSpec((1, _O), lambda b: (0, 0)),
            pl.BlockSpec((1, _O), lambda b: (0, 0)),
        ],
        out_specs=pl.BlockSpec((_S, _N, _O), lambda b: (b, 0, 0)),
        out_shape=jax.ShapeDtypeStruct((_B, _N, _O), jnp.float32),
        compiler_params=pltpu.CompilerParams(
            dimension_semantics=("parallel",)),
    )(x, W, a_src.reshape(1, _O), a_dst.reshape(1, _O), bias.reshape(1, _O))
    return out
